# Initial kernel scaffold; baseline (speedup 1.0000x reference)
import jax
import jax.numpy as jnp
from jax.experimental import pallas as pl


def kernel(x, edge_index, batch, W1, b1, W2, b2, W3, b3, lin1_W, lin1_b, lin2_W, lin2_b):
    raise NotImplementedError("write your pallas kernel here")


# speedup vs baseline: 16.8660x; 16.8660x over previous
import functools

import jax
import jax.numpy as jnp
from jax import lax
from jax.experimental import pallas as pl
from jax.experimental.pallas import tpu as pltpu
from jax.experimental.pallas import tpu_sc as plsc

N = 10000
E = 320000
D = 128
H = 128
G = 128
OUT = 16

N_PAD = 10240
BLK = 1024
NBLK = N_PAD // BLK

NC = 2
NS = 16
NW = NC * NS
EPW = E // NW
CH = 80
NCH = EPW // CH
RPT = N_PAD // NS

@functools.lru_cache(maxsize=1)
def _sc_mesh():
    return plsc.VectorSubcoreMesh(core_axis_name="c", subcore_axis_name="s",
                                  num_cores=NC, num_subcores=NS)


def _deg_body(col2d_hbm, zeros1_hbm, deg_hbm, acc_sh, colslab, onesv, sem):
    del sem
    c = lax.axis_index("c")
    s = lax.axis_index("s")
    wid = c * NS + s
    pltpu.sync_copy(zeros1_hbm.at[pl.ds(s * RPT, RPT)], acc_sh.at[pl.ds(s * RPT, RPT)])
    pltpu.sync_copy(col2d_hbm.at[wid], colslab)
    for k in range(CH // 16):
        onesv[pl.ds(k * 16, 16)] = jnp.ones((16,), jnp.float32)
    plsc.subcore_barrier()

    def chunk(j, carry):
        pltpu.sync_copy(onesv, acc_sh.at[colslab.at[j]], add=True)
        return carry

    lax.fori_loop(0, NCH, chunk, 0)
    plsc.subcore_barrier()
    pltpu.sync_copy(acc_sh.at[pl.ds(s * RPT, RPT)], deg_hbm.at[c, pl.ds(s * RPT, RPT)])


@functools.lru_cache(maxsize=1)
def _deg_call():
    return pl.kernel(
        _deg_body,
        out_type=jax.ShapeDtypeStruct((NC, N_PAD), jnp.float32),
        mesh=_sc_mesh(),
        scratch_types=[
            pltpu.VMEM_SHARED((N_PAD,), jnp.float32),
            pltpu.VMEM((NCH, CH), jnp.int32),
            pltpu.VMEM((CH,), jnp.float32),
            pltpu.SemaphoreType.DMA,
        ],
    )


def _agg_body(u_hbm, row2d_hbm, col2d_hbm, zeros2_hbm, p_hbm,
              acc_sh, rowslab, colslab, rowsv, sem):
    c = lax.axis_index("c")
    s = lax.axis_index("s")
    wid = c * NS + s
    pltpu.sync_copy(zeros2_hbm, acc_sh.at[pl.ds(s * RPT, RPT)])
    pltpu.sync_copy(row2d_hbm.at[wid], rowslab)
    pltpu.sync_copy(col2d_hbm.at[wid], colslab)
    plsc.subcore_barrier()

    def chunk(j, carry):
        pltpu.async_copy(u_hbm.at[rowslab.at[j]], rowsv, sem).wait()
        pltpu.sync_copy(rowsv, acc_sh.at[colslab.at[j]], add=True)
        return carry

    lax.fori_loop(0, NCH, chunk, 0)
    plsc.subcore_barrier()
    pltpu.sync_copy(acc_sh.at[pl.ds(s * RPT, RPT)], p_hbm.at[c, pl.ds(s * RPT, RPT)])


@functools.lru_cache(maxsize=1)
def _agg_call():
    return pl.kernel(
        _agg_body,
        out_type=jax.ShapeDtypeStruct((NC, N_PAD, H), jnp.float32),
        mesh=_sc_mesh(),
        scratch_types=[
            pltpu.VMEM_SHARED((N_PAD, H), jnp.float32),
            pltpu.VMEM((NCH, CH), jnp.int32),
            pltpu.VMEM((NCH, CH), jnp.int32),
            pltpu.VMEM((CH, H), jnp.float32),
            pltpu.SemaphoreType.DMA,
        ],
    )


def _dinv_from(deg_ref):
    deg = 1.0 + jnp.sum(deg_ref[...], axis=1, keepdims=True)
    return lax.rsqrt(deg)


def _onehot(b_ref):
    bb = b_ref[0]
    gid = lax.broadcasted_iota(jnp.int32, (G, BLK), 0)
    return (jnp.broadcast_to(bb, (G, BLK)) == gid).astype(jnp.float32)


def _first_body(x_ref, w_ref, deg_ref, b_ref, u_ref, cnt_ref):
    i = pl.program_id(0)
    dinv = _dinv_from(deg_ref)
    xw = jnp.dot(x_ref[...], w_ref[...], preferred_element_type=jnp.float32)
    u_ref[...] = dinv * xw
    oh = _onehot(b_ref)

    @pl.when(i == 0)
    def _():
        cnt_ref[...] = jnp.zeros_like(cnt_ref)

    cnt_ref[...] += jnp.dot(oh, jnp.ones((BLK, G), jnp.float32),
                            preferred_element_type=jnp.float32)


def _first_call(x_pad, W, deg2, batch3):
    return pl.pallas_call(
        _first_body,
        grid=(NBLK,),
        in_specs=[
            pl.BlockSpec((BLK, D), lambda i: (i, 0)),
            pl.BlockSpec((D, H), lambda i: (0, 0)),
            pl.BlockSpec((BLK, 2), lambda i: (i, 0)),
            pl.BlockSpec((1, 1, BLK), lambda i: (i, 0, 0)),
        ],
        out_specs=[
            pl.BlockSpec((BLK, H), lambda i: (i, 0)),
            pl.BlockSpec((G, G), lambda i: (0, 0)),
        ],
        out_shape=[
            jax.ShapeDtypeStruct((N_PAD, H), jnp.float32),
            jax.ShapeDtypeStruct((G, G), jnp.float32),
        ],
    )(x_pad, W, deg2, batch3)


def _layer_body(p_ref, u_ref, deg_ref, bias_ref, w_ref, b_ref,
                unext_ref, pool_ref):
    i = pl.program_id(0)
    dinv = _dinv_from(deg_ref)
    agg = p_ref[0] + p_ref[1]
    h = dinv * (agg + u_ref[...]) + bias_ref[...]
    unext_ref[...] = dinv * jnp.dot(h, w_ref[...], preferred_element_type=jnp.float32)
    oh = _onehot(b_ref)

    @pl.when(i == 0)
    def _():
        pool_ref[...] = jnp.zeros_like(pool_ref)

    pool_ref[...] += jnp.dot(oh, h, preferred_element_type=jnp.float32)


def _layer_call(p, u, deg2, bias, W, batch3):
    return pl.pallas_call(
        _layer_body,
        grid=(NBLK,),
        in_specs=[
            pl.BlockSpec((NC, BLK, H), lambda i: (0, i, 0)),
            pl.BlockSpec((BLK, H), lambda i: (i, 0)),
            pl.BlockSpec((BLK, 2), lambda i: (i, 0)),
            pl.BlockSpec((1, H), lambda i: (0, 0)),
            pl.BlockSpec((H, H), lambda i: (0, 0)),
            pl.BlockSpec((1, 1, BLK), lambda i: (i, 0, 0)),
        ],
        out_specs=[
            pl.BlockSpec((BLK, H), lambda i: (i, 0)),
            pl.BlockSpec((G, H), lambda i: (0, 0)),
        ],
        out_shape=[
            jax.ShapeDtypeStruct((N_PAD, H), jnp.float32),
            jax.ShapeDtypeStruct((G, H), jnp.float32),
        ],
    )(p, u, deg2, bias, W, batch3)


def _last_body(p_ref, u_ref, deg_ref, bias_ref, b_ref, pool_ref):
    i = pl.program_id(0)
    dinv = _dinv_from(deg_ref)
    agg = p_ref[0] + p_ref[1]
    h = dinv * (agg + u_ref[...]) + bias_ref[...]
    oh = _onehot(b_ref)

    @pl.when(i == 0)
    def _():
        pool_ref[...] = jnp.zeros_like(pool_ref)

    pool_ref[...] += jnp.dot(oh, h, preferred_element_type=jnp.float32)


def _last_call(p, u, deg2, bias, batch3):
    return pl.pallas_call(
        _last_body,
        grid=(NBLK,),
        in_specs=[
            pl.BlockSpec((NC, BLK, H), lambda i: (0, i, 0)),
            pl.BlockSpec((BLK, H), lambda i: (i, 0)),
            pl.BlockSpec((BLK, 2), lambda i: (i, 0)),
            pl.BlockSpec((1, H), lambda i: (0, 0)),
            pl.BlockSpec((1, 1, BLK), lambda i: (i, 0, 0)),
        ],
        out_specs=pl.BlockSpec((G, H), lambda i: (0, 0)),
        out_shape=jax.ShapeDtypeStruct((G, H), jnp.float32),
    )(p, u, deg2, bias, batch3)


def _head_body(p1_ref, p2_ref, p3_ref, cnt_ref, w1_ref, b1_ref, w2_ref, b2_ref,
               h_ref, ls_ref):
    cnt = jnp.maximum(cnt_ref[...][:, 0:1], 1.0)
    poolcat = jnp.concatenate([p1_ref[...], p2_ref[...], p3_ref[...]], axis=1) / cnt
    t = jnp.dot(poolcat, w1_ref[...], preferred_element_type=jnp.float32) + b1_ref[...]
    t = jnp.maximum(t, 0.0)
    o = jnp.dot(t, w2_ref[...], preferred_element_type=jnp.float32) + b2_ref[...]
    h_ref[...] = o
    colid = lax.broadcasted_iota(jnp.int32, (G, 128), 1)
    valid = colid < OUT
    om = jnp.where(valid, o, -1e30)
    m = jnp.max(om, axis=1, keepdims=True)
    ssum = jnp.sum(jnp.where(valid, jnp.exp(om - m), 0.0), axis=1, keepdims=True)
    ls_ref[...] = om - m - jnp.log(ssum)


def _head_call(pool1, pool2, pool3, counts, lin1_W, lin1_b, lin2_Wp, lin2_bp):
    HH = 3 * H
    return pl.pallas_call(
        _head_body,
        in_specs=[
            pl.BlockSpec((G, H), lambda: (0, 0)),
            pl.BlockSpec((G, H), lambda: (0, 0)),
            pl.BlockSpec((G, H), lambda: (0, 0)),
            pl.BlockSpec((G, G), lambda: (0, 0)),
            pl.BlockSpec((HH, HH), lambda: (0, 0)),
            pl.BlockSpec((1, HH), lambda: (0, 0)),
            pl.BlockSpec((HH, 128), lambda: (0, 0)),
            pl.BlockSpec((1, 128), lambda: (0, 0)),
        ],
        out_specs=[
            pl.BlockSpec((G, 128), lambda: (0, 0)),
            pl.BlockSpec((G, 128), lambda: (0, 0)),
        ],
        out_shape=[
            jax.ShapeDtypeStruct((G, 128), jnp.float32),
            jax.ShapeDtypeStruct((G, 128), jnp.float32),
        ],
    )(pool1, pool2, pool3, counts, lin1_W, lin1_b, lin2_Wp, lin2_bp)


def kernel(x, edge_index, batch, W1, b1, W2, b2, W3, b3,
           lin1_W, lin1_b, lin2_W, lin2_b):
    f32 = jnp.float32
    row2d = edge_index[0].reshape(NW, NCH, CH)
    col2d = edge_index[1].reshape(NW, NCH, CH)
    x_pad = jnp.pad(x, ((0, N_PAD - N), (0, 0)))
    batch_pad = jnp.concatenate([batch, jnp.full((N_PAD - N,), G, jnp.int32)])
    batch3 = batch_pad.reshape(NBLK, 1, BLK)
    zeros1 = jnp.zeros((N_PAD,), f32)
    zeros2 = jnp.zeros((RPT, H), f32)

    degT = _deg_call()(col2d, zeros1)
    deg2 = degT.T

    u1, counts = _first_call(x_pad, W1, deg2, batch3)
    p = _agg_call()(u1, row2d, col2d, zeros2)
    u2, pool1 = _layer_call(p, u1, deg2, b1.reshape(1, H), W2, batch3)
    p = _agg_call()(u2, row2d, col2d, zeros2)
    u3, pool2 = _layer_call(p, u2, deg2, b2.reshape(1, H), W3, batch3)
    p = _agg_call()(u3, row2d, col2d, zeros2)
    pool3 = _last_call(p, u3, deg2, b3.reshape(1, H), batch3)

    lin2_Wp = jnp.pad(lin2_W, ((0, 0), (0, 128 - OUT)))
    lin2_bp = jnp.pad(lin2_b, (0, 128 - OUT)).reshape(1, 128)
    hout, ls = _head_call(pool1, pool2, pool3, counts,
                          lin1_W, lin1_b.reshape(1, 3 * H), lin2_Wp, lin2_bp)
    return (hout[:, :OUT], ls[:, :OUT])


# speedup vs baseline: 22.7709x; 1.3501x over previous
import functools

import jax
import jax.numpy as jnp
from jax import lax
from jax.experimental import pallas as pl
from jax.experimental.pallas import tpu as pltpu
from jax.experimental.pallas import tpu_sc as plsc

N = 10000
E = 320000
D = 128
H = 128
G = 128
OUT = 16

N_PAD = 10240
BLK = 1024
NBLK = N_PAD // BLK

NC = 2
NS = 16
NW = NC * NS
EPW = E // NW
CH = 80
NCH = EPW // CH
RPT = N_PAD // NS

NBUF = 3
NGRP = NCH // NBUF
NTAIL = NCH - NGRP * NBUF


@functools.lru_cache(maxsize=1)
def _sc_mesh():
    return plsc.VectorSubcoreMesh(core_axis_name="c", subcore_axis_name="s",
                                  num_cores=NC, num_subcores=NS)


def _deg_body(col2d_hbm, zeros1_hbm, deg_hbm, acc_sh, colslab, onesv, sem):
    del sem
    c = lax.axis_index("c")
    s = lax.axis_index("s")
    wid = c * NS + s
    pltpu.sync_copy(zeros1_hbm.at[pl.ds(s * RPT, RPT)], acc_sh.at[pl.ds(s * RPT, RPT)])
    pltpu.sync_copy(col2d_hbm.at[wid], colslab)
    for k in range(CH // 16):
        onesv[pl.ds(k * 16, 16)] = jnp.ones((16,), jnp.float32)
    plsc.subcore_barrier()

    def chunk(j, carry):
        pltpu.sync_copy(onesv, acc_sh.at[colslab.at[j]], add=True)
        return carry

    lax.fori_loop(0, NCH, chunk, 0)
    plsc.subcore_barrier()
    pltpu.sync_copy(acc_sh.at[pl.ds(s * RPT, RPT)], deg_hbm.at[c, pl.ds(s * RPT, RPT)])


@functools.lru_cache(maxsize=1)
def _deg_call():
    return pl.kernel(
        _deg_body,
        out_type=jax.ShapeDtypeStruct((NC, N_PAD), jnp.float32),
        mesh=_sc_mesh(),
        scratch_types=[
            pltpu.VMEM_SHARED((N_PAD,), jnp.float32),
            pltpu.VMEM((NCH, CH), jnp.int32),
            pltpu.VMEM((CH,), jnp.float32),
            pltpu.SemaphoreType.DMA,
        ],
    )


def _agg_body(u_hbm, row2d_hbm, col4d_hbm, zeros2_hbm, p_hbm,
              acc_sh, rowslab, rowsv, coltmp, gsems, csems):
    c = lax.axis_index("c")
    s = lax.axis_index("s")
    wid = c * NS + s
    pltpu.sync_copy(zeros2_hbm, acc_sh.at[pl.ds(s * RPT, RPT)])
    pltpu.sync_copy(row2d_hbm.at[wid], rowslab)
    plsc.subcore_barrier()

    def fire(j, b):
        dg = pltpu.async_copy(u_hbm.at[rowslab.at[j]], rowsv.at[b], gsems.at[b])
        dc = pltpu.async_copy(col4d_hbm.at[wid, j], coltmp.at[b], csems.at[b])
        return dg, dc

    def drain(descs, b):
        dg, dc = descs
        dc.wait()
        dg.wait()
        pltpu.sync_copy(rowsv.at[b], acc_sh.at[coltmp.at[b, 0]], add=True)

    def outer(g, carry):
        j0 = g * NBUF
        descs = [fire(j0 + b, b) for b in range(NBUF)]
        for b in range(NBUF):
            drain(descs[b], b)
        return carry

    lax.fori_loop(0, NGRP, outer, 0)
    descs = [fire(NGRP * NBUF + b, b) for b in range(NTAIL)]
    for b in range(NTAIL):
        drain(descs[b], b)
    plsc.subcore_barrier()
    pltpu.sync_copy(acc_sh.at[pl.ds(s * RPT, RPT)], p_hbm.at[c, pl.ds(s * RPT, RPT)])


@functools.lru_cache(maxsize=1)
def _agg_call():
    return pl.kernel(
        _agg_body,
        out_type=jax.ShapeDtypeStruct((NC, N_PAD, H), jnp.float32),
        mesh=_sc_mesh(),
        scratch_types=[
            pltpu.VMEM_SHARED((N_PAD, H), jnp.float32),
            pltpu.VMEM((NCH, CH), jnp.int32),
            pltpu.VMEM((NBUF, CH, H), jnp.float32),
            pltpu.VMEM((NBUF, 1, CH), jnp.int32),
            pltpu.SemaphoreType.DMA((NBUF,)),
            pltpu.SemaphoreType.DMA((NBUF,)),
        ],
    )


def _dinv_from(deg_ref):
    deg = 1.0 + jnp.sum(deg_ref[...], axis=1, keepdims=True)
    return lax.rsqrt(deg)


def _onehot(b_ref):
    bb = b_ref[0]
    gid = lax.broadcasted_iota(jnp.int32, (G, BLK), 0)
    return (jnp.broadcast_to(bb, (G, BLK)) == gid).astype(jnp.float32)


def _first_body(x_ref, w_ref, deg_ref, b_ref, u_ref, cnt_ref):
    i = pl.program_id(0)
    dinv = _dinv_from(deg_ref)
    xw = jnp.dot(x_ref[...], w_ref[...], preferred_element_type=jnp.float32)
    u_ref[...] = dinv * xw
    oh = _onehot(b_ref)

    @pl.when(i == 0)
    def _():
        cnt_ref[...] = jnp.zeros_like(cnt_ref)

    cnt_ref[...] += jnp.dot(oh, jnp.ones((BLK, G), jnp.float32),
                            preferred_element_type=jnp.float32)


def _first_call(x, W, deg2, batch3):
    return pl.pallas_call(
        _first_body,
        grid=(NBLK,),
        in_specs=[
            pl.BlockSpec((BLK, D), lambda i: (i, 0)),
            pl.BlockSpec((D, H), lambda i: (0, 0)),
            pl.BlockSpec((BLK, 2), lambda i: (i, 0)),
            pl.BlockSpec((1, 1, BLK), lambda i: (i, 0, 0)),
        ],
        out_specs=[
            pl.BlockSpec((BLK, H), lambda i: (i, 0)),
            pl.BlockSpec((G, G), lambda i: (0, 0)),
        ],
        out_shape=[
            jax.ShapeDtypeStruct((N_PAD, H), jnp.float32),
            jax.ShapeDtypeStruct((G, G), jnp.float32),
        ],
    )(x, W, deg2, batch3)


def _layer_body(p_ref, u_ref, deg_ref, bias_ref, w_ref, b_ref,
                unext_ref, pool_ref):
    i = pl.program_id(0)
    dinv = _dinv_from(deg_ref)
    agg = p_ref[0] + p_ref[1]
    h = dinv * (agg + u_ref[...]) + bias_ref[...]
    unext_ref[...] = dinv * jnp.dot(h, w_ref[...], preferred_element_type=jnp.float32)
    oh = _onehot(b_ref)

    @pl.when(i == 0)
    def _():
        pool_ref[...] = jnp.zeros_like(pool_ref)

    pool_ref[...] += jnp.dot(oh, h, preferred_element_type=jnp.float32)


def _layer_call(p, u, deg2, bias, W, batch3):
    return pl.pallas_call(
        _layer_body,
        grid=(NBLK,),
        in_specs=[
            pl.BlockSpec((NC, BLK, H), lambda i: (0, i, 0)),
            pl.BlockSpec((BLK, H), lambda i: (i, 0)),
            pl.BlockSpec((BLK, 2), lambda i: (i, 0)),
            pl.BlockSpec((1, H), lambda i: (0, 0)),
            pl.BlockSpec((H, H), lambda i: (0, 0)),
            pl.BlockSpec((1, 1, BLK), lambda i: (i, 0, 0)),
        ],
        out_specs=[
            pl.BlockSpec((BLK, H), lambda i: (i, 0)),
            pl.BlockSpec((G, H), lambda i: (0, 0)),
        ],
        out_shape=[
            jax.ShapeDtypeStruct((N_PAD, H), jnp.float32),
            jax.ShapeDtypeStruct((G, H), jnp.float32),
        ],
    )(p, u, deg2, bias, W, batch3)


def _last_body(p_ref, u_ref, deg_ref, bias_ref, b_ref, pool_ref):
    i = pl.program_id(0)
    dinv = _dinv_from(deg_ref)
    agg = p_ref[0] + p_ref[1]
    h = dinv * (agg + u_ref[...]) + bias_ref[...]
    oh = _onehot(b_ref)

    @pl.when(i == 0)
    def _():
        pool_ref[...] = jnp.zeros_like(pool_ref)

    pool_ref[...] += jnp.dot(oh, h, preferred_element_type=jnp.float32)


def _last_call(p, u, deg2, bias, batch3):
    return pl.pallas_call(
        _last_body,
        grid=(NBLK,),
        in_specs=[
            pl.BlockSpec((NC, BLK, H), lambda i: (0, i, 0)),
            pl.BlockSpec((BLK, H), lambda i: (i, 0)),
            pl.BlockSpec((BLK, 2), lambda i: (i, 0)),
            pl.BlockSpec((1, H), lambda i: (0, 0)),
            pl.BlockSpec((1, 1, BLK), lambda i: (i, 0, 0)),
        ],
        out_specs=pl.BlockSpec((G, H), lambda i: (0, 0)),
        out_shape=jax.ShapeDtypeStruct((G, H), jnp.float32),
    )(p, u, deg2, bias, batch3)


def _head_body(p1_ref, p2_ref, p3_ref, cnt_ref, w1_ref, b1_ref, w2_ref, b2_ref,
               h_ref, ls_ref):
    cnt = jnp.maximum(cnt_ref[...][:, 0:1], 1.0)
    poolcat = jnp.concatenate([p1_ref[...], p2_ref[...], p3_ref[...]], axis=1) / cnt
    t = jnp.dot(poolcat, w1_ref[...], preferred_element_type=jnp.float32) + b1_ref[...]
    t = jnp.maximum(t, 0.0)
    o = jnp.dot(t, w2_ref[...], preferred_element_type=jnp.float32) + b2_ref[...]
    h_ref[...] = o
    colid = lax.broadcasted_iota(jnp.int32, (G, 128), 1)
    valid = colid < OUT
    om = jnp.where(valid, o, -1e30)
    m = jnp.max(om, axis=1, keepdims=True)
    ssum = jnp.sum(jnp.where(valid, jnp.exp(om - m), 0.0), axis=1, keepdims=True)
    ls_ref[...] = om - m - jnp.log(ssum)


def _head_call(pool1, pool2, pool3, counts, lin1_W, lin1_b, lin2_Wp, lin2_bp):
    HH = 3 * H
    return pl.pallas_call(
        _head_body,
        in_specs=[
            pl.BlockSpec((G, H), lambda: (0, 0)),
            pl.BlockSpec((G, H), lambda: (0, 0)),
            pl.BlockSpec((G, H), lambda: (0, 0)),
            pl.BlockSpec((G, G), lambda: (0, 0)),
            pl.BlockSpec((HH, HH), lambda: (0, 0)),
            pl.BlockSpec((1, HH), lambda: (0, 0)),
            pl.BlockSpec((HH, 128), lambda: (0, 0)),
            pl.BlockSpec((1, 128), lambda: (0, 0)),
        ],
        out_specs=[
            pl.BlockSpec((G, 128), lambda: (0, 0)),
            pl.BlockSpec((G, 128), lambda: (0, 0)),
        ],
        out_shape=[
            jax.ShapeDtypeStruct((G, 128), jnp.float32),
            jax.ShapeDtypeStruct((G, 128), jnp.float32),
        ],
    )(pool1, pool2, pool3, counts, lin1_W, lin1_b, lin2_Wp, lin2_bp)


def kernel(x, edge_index, batch, W1, b1, W2, b2, W3, b3,
           lin1_W, lin1_b, lin2_W, lin2_b):
    f32 = jnp.float32
    row2d = edge_index[0].reshape(NW, NCH, CH)
    col2d = edge_index[1].reshape(NW, NCH, CH)
    col4d = edge_index[1].reshape(NW, NCH, 1, CH)
    x_pad = jnp.pad(x, ((0, N_PAD - N), (0, 0)))
    batch_pad = jnp.concatenate([batch, jnp.full((N_PAD - N,), G, jnp.int32)])
    batch3 = batch_pad.reshape(NBLK, 1, BLK)
    zeros1 = jnp.zeros((N_PAD,), f32)
    zeros2 = jnp.zeros((RPT, H), f32)

    degT = _deg_call()(col2d, zeros1)
    deg2 = degT.T

    u1, counts = _first_call(x_pad, W1, deg2, batch3)
    p = _agg_call()(u1, row2d, col4d, zeros2)
    u2, pool1 = _layer_call(p, u1, deg2, b1.reshape(1, H), W2, batch3)
    p = _agg_call()(u2, row2d, col4d, zeros2)
    u3, pool2 = _layer_call(p, u2, deg2, b2.reshape(1, H), W3, batch3)
    p = _agg_call()(u3, row2d, col4d, zeros2)
    pool3 = _last_call(p, u3, deg2, b3.reshape(1, H), batch3)

    lin2_Wp = jnp.pad(lin2_W, ((0, 0), (0, 128 - OUT)))
    lin2_bp = jnp.pad(lin2_b, (0, 128 - OUT)).reshape(1, 128)
    hout, ls = _head_call(pool1, pool2, pool3, counts,
                          lin1_W, lin1_b.reshape(1, 3 * H), lin2_Wp, lin2_bp)
    return (hout[:, :OUT], ls[:, :OUT])


# speedup vs baseline: 23.3799x; 1.0267x over previous
import functools

import jax
import jax.numpy as jnp
from jax import lax
from jax.experimental import pallas as pl
from jax.experimental.pallas import tpu as pltpu
from jax.experimental.pallas import tpu_sc as plsc

N = 10000
E = 320000
D = 128
H = 128
G = 128
OUT = 16

N_PAD = 10240
BLK = 1024
NBLK = N_PAD // BLK

NC = 2
NS = 16
NW = NC * NS
EPW = E // NW
CH = 80
NCH = EPW // CH
RPT = N_PAD // NS

NBUF = 3
NGRP = NCH // NBUF
NTAIL = NCH - NGRP * NBUF


@functools.lru_cache(maxsize=1)
def _sc_mesh():
    return plsc.VectorSubcoreMesh(core_axis_name="c", subcore_axis_name="s",
                                  num_cores=NC, num_subcores=NS)


def _deg_body(col2d_hbm, zeros1_hbm, deg_hbm, acc_sh, colslab, onesv, sem):
    del sem
    c = lax.axis_index("c")
    s = lax.axis_index("s")
    wid = c * NS + s
    pltpu.sync_copy(zeros1_hbm.at[pl.ds(s * RPT, RPT)], acc_sh.at[pl.ds(s * RPT, RPT)])
    pltpu.sync_copy(col2d_hbm.at[wid], colslab)
    for k in range(CH // 16):
        onesv[pl.ds(k * 16, 16)] = jnp.ones((16,), jnp.float32)
    plsc.subcore_barrier()

    def chunk(j, carry):
        pltpu.sync_copy(onesv, acc_sh.at[colslab.at[j]], add=True)
        return carry

    lax.fori_loop(0, NCH, chunk, 0)
    plsc.subcore_barrier()
    pltpu.sync_copy(acc_sh.at[pl.ds(s * RPT, RPT)], deg_hbm.at[c, pl.ds(s * RPT, RPT)])


@functools.lru_cache(maxsize=1)
def _deg_call():
    return pl.kernel(
        _deg_body,
        out_type=jax.ShapeDtypeStruct((NC, N_PAD), jnp.float32),
        mesh=_sc_mesh(),
        scratch_types=[
            pltpu.VMEM_SHARED((N_PAD,), jnp.float32),
            pltpu.VMEM((NCH, CH), jnp.int32),
            pltpu.VMEM((CH,), jnp.float32),
            pltpu.SemaphoreType.DMA,
        ],
    )


def _agg_body(u_hbm, row2d_hbm, col4d_hbm, zeros2_hbm, p_hbm,
              acc_sh, rowslab, rowsv, coltmp, gsems, csems, ssems):
    c = lax.axis_index("c")
    s = lax.axis_index("s")
    wid = c * NS + s
    pltpu.sync_copy(zeros2_hbm, acc_sh.at[pl.ds(s * RPT, RPT)])
    pltpu.sync_copy(row2d_hbm.at[wid], rowslab)
    plsc.subcore_barrier()

    def fire(j, b):
        dg = pltpu.async_copy(u_hbm.at[rowslab.at[j]], rowsv.at[b], gsems.at[b])
        dc = pltpu.async_copy(col4d_hbm.at[wid, j], coltmp.at[b], csems.at[b])
        return dg, dc

    def scat(descs, b):
        dg, dc = descs
        dc.wait()
        dg.wait()
        return pltpu.async_copy(rowsv.at[b], acc_sh.at[coltmp.at[b, 0]],
                                ssems.at[b], add=True)

    def outer(g, carry):
        j0 = g * NBUF
        descs = [fire(j0 + b, b) for b in range(NBUF)]
        sdescs = [scat(descs[b], b) for b in range(NBUF)]
        for b in range(NBUF):
            sdescs[b].wait()
        return carry

    lax.fori_loop(0, NGRP, outer, 0)
    descs = [fire(NGRP * NBUF + b, b) for b in range(NTAIL)]
    sdescs = [scat(descs[b], b) for b in range(NTAIL)]
    for b in range(NTAIL):
        sdescs[b].wait()
    plsc.subcore_barrier()
    pltpu.sync_copy(acc_sh.at[pl.ds(s * RPT, RPT)], p_hbm.at[c, pl.ds(s * RPT, RPT)])


@functools.lru_cache(maxsize=1)
def _agg_call():
    return pl.kernel(
        _agg_body,
        out_type=jax.ShapeDtypeStruct((NC, N_PAD, H), jnp.float32),
        mesh=_sc_mesh(),
        scratch_types=[
            pltpu.VMEM_SHARED((N_PAD, H), jnp.float32),
            pltpu.VMEM((NCH, CH), jnp.int32),
            pltpu.VMEM((NBUF, CH, H), jnp.float32),
            pltpu.VMEM((NBUF, 1, CH), jnp.int32),
            pltpu.SemaphoreType.DMA((NBUF,)),
            pltpu.SemaphoreType.DMA((NBUF,)),
            pltpu.SemaphoreType.DMA((NBUF,)),
        ],
    )


def _dinv_from(deg_ref):
    deg = 1.0 + jnp.sum(deg_ref[...], axis=1, keepdims=True)
    return lax.rsqrt(deg)


def _onehot(b_ref):
    bb = b_ref[0]
    gid = lax.broadcasted_iota(jnp.int32, (G, BLK), 0)
    return (jnp.broadcast_to(bb, (G, BLK)) == gid).astype(jnp.float32)


def _first_body(x_ref, w_ref, deg_ref, b_ref, u_ref, cnt_ref):
    i = pl.program_id(0)
    dinv = _dinv_from(deg_ref)
    xw = jnp.dot(x_ref[...], w_ref[...], preferred_element_type=jnp.float32)
    u_ref[...] = dinv * xw
    oh = _onehot(b_ref)

    @pl.when(i == 0)
    def _():
        cnt_ref[...] = jnp.zeros_like(cnt_ref)

    cnt_ref[...] += jnp.dot(oh, jnp.ones((BLK, G), jnp.float32),
                            preferred_element_type=jnp.float32)


def _first_call(x, W, deg2, batch3):
    return pl.pallas_call(
        _first_body,
        grid=(NBLK,),
        in_specs=[
            pl.BlockSpec((BLK, D), lambda i: (i, 0)),
            pl.BlockSpec((D, H), lambda i: (0, 0)),
            pl.BlockSpec((BLK, 2), lambda i: (i, 0)),
            pl.BlockSpec((1, 1, BLK), lambda i: (i, 0, 0)),
        ],
        out_specs=[
            pl.BlockSpec((BLK, H), lambda i: (i, 0)),
            pl.BlockSpec((G, G), lambda i: (0, 0)),
        ],
        out_shape=[
            jax.ShapeDtypeStruct((N_PAD, H), jnp.float32),
            jax.ShapeDtypeStruct((G, G), jnp.float32),
        ],
    )(x, W, deg2, batch3)


def _layer_body(p_ref, u_ref, deg_ref, bias_ref, w_ref, b_ref,
                unext_ref, pool_ref):
    i = pl.program_id(0)
    dinv = _dinv_from(deg_ref)
    agg = p_ref[0] + p_ref[1]
    h = dinv * (agg + u_ref[...]) + bias_ref[...]
    unext_ref[...] = dinv * jnp.dot(h, w_ref[...], preferred_element_type=jnp.float32)
    oh = _onehot(b_ref)

    @pl.when(i == 0)
    def _():
        pool_ref[...] = jnp.zeros_like(pool_ref)

    pool_ref[...] += jnp.dot(oh, h, preferred_element_type=jnp.float32)


def _layer_call(p, u, deg2, bias, W, batch3):
    return pl.pallas_call(
        _layer_body,
        grid=(NBLK,),
        in_specs=[
            pl.BlockSpec((NC, BLK, H), lambda i: (0, i, 0)),
            pl.BlockSpec((BLK, H), lambda i: (i, 0)),
            pl.BlockSpec((BLK, 2), lambda i: (i, 0)),
            pl.BlockSpec((1, H), lambda i: (0, 0)),
            pl.BlockSpec((H, H), lambda i: (0, 0)),
            pl.BlockSpec((1, 1, BLK), lambda i: (i, 0, 0)),
        ],
        out_specs=[
            pl.BlockSpec((BLK, H), lambda i: (i, 0)),
            pl.BlockSpec((G, H), lambda i: (0, 0)),
        ],
        out_shape=[
            jax.ShapeDtypeStruct((N_PAD, H), jnp.float32),
            jax.ShapeDtypeStruct((G, H), jnp.float32),
        ],
    )(p, u, deg2, bias, W, batch3)


def _last_body(p_ref, u_ref, deg_ref, bias_ref, b_ref, pool_ref):
    i = pl.program_id(0)
    dinv = _dinv_from(deg_ref)
    agg = p_ref[0] + p_ref[1]
    h = dinv * (agg + u_ref[...]) + bias_ref[...]
    oh = _onehot(b_ref)

    @pl.when(i == 0)
    def _():
        pool_ref[...] = jnp.zeros_like(pool_ref)

    pool_ref[...] += jnp.dot(oh, h, preferred_element_type=jnp.float32)


def _last_call(p, u, deg2, bias, batch3):
    return pl.pallas_call(
        _last_body,
        grid=(NBLK,),
        in_specs=[
            pl.BlockSpec((NC, BLK, H), lambda i: (0, i, 0)),
            pl.BlockSpec((BLK, H), lambda i: (i, 0)),
            pl.BlockSpec((BLK, 2), lambda i: (i, 0)),
            pl.BlockSpec((1, H), lambda i: (0, 0)),
            pl.BlockSpec((1, 1, BLK), lambda i: (i, 0, 0)),
        ],
        out_specs=pl.BlockSpec((G, H), lambda i: (0, 0)),
        out_shape=jax.ShapeDtypeStruct((G, H), jnp.float32),
    )(p, u, deg2, bias, batch3)


def _head_body(p1_ref, p2_ref, p3_ref, cnt_ref, w1_ref, b1_ref, w2_ref, b2_ref,
               h_ref, ls_ref):
    cnt = jnp.maximum(cnt_ref[...][:, 0:1], 1.0)
    poolcat = jnp.concatenate([p1_ref[...], p2_ref[...], p3_ref[...]], axis=1) / cnt
    t = jnp.dot(poolcat, w1_ref[...], preferred_element_type=jnp.float32) + b1_ref[...]
    t = jnp.maximum(t, 0.0)
    o = jnp.dot(t, w2_ref[...], preferred_element_type=jnp.float32) + b2_ref[...]
    h_ref[...] = o
    colid = lax.broadcasted_iota(jnp.int32, (G, 128), 1)
    valid = colid < OUT
    om = jnp.where(valid, o, -1e30)
    m = jnp.max(om, axis=1, keepdims=True)
    ssum = jnp.sum(jnp.where(valid, jnp.exp(om - m), 0.0), axis=1, keepdims=True)
    ls_ref[...] = om - m - jnp.log(ssum)


def _head_call(pool1, pool2, pool3, counts, lin1_W, lin1_b, lin2_Wp, lin2_bp):
    HH = 3 * H
    return pl.pallas_call(
        _head_body,
        in_specs=[
            pl.BlockSpec((G, H), lambda: (0, 0)),
            pl.BlockSpec((G, H), lambda: (0, 0)),
            pl.BlockSpec((G, H), lambda: (0, 0)),
            pl.BlockSpec((G, G), lambda: (0, 0)),
            pl.BlockSpec((HH, HH), lambda: (0, 0)),
            pl.BlockSpec((1, HH), lambda: (0, 0)),
            pl.BlockSpec((HH, 128), lambda: (0, 0)),
            pl.BlockSpec((1, 128), lambda: (0, 0)),
        ],
        out_specs=[
            pl.BlockSpec((G, 128), lambda: (0, 0)),
            pl.BlockSpec((G, 128), lambda: (0, 0)),
        ],
        out_shape=[
            jax.ShapeDtypeStruct((G, 128), jnp.float32),
            jax.ShapeDtypeStruct((G, 128), jnp.float32),
        ],
    )(pool1, pool2, pool3, counts, lin1_W, lin1_b, lin2_Wp, lin2_bp)


def kernel(x, edge_index, batch, W1, b1, W2, b2, W3, b3,
           lin1_W, lin1_b, lin2_W, lin2_b):
    f32 = jnp.float32
    row2d = edge_index[0].reshape(NW, NCH, CH)
    col2d = edge_index[1].reshape(NW, NCH, CH)
    col4d = edge_index[1].reshape(NW, NCH, 1, CH)
    x_pad = jnp.pad(x, ((0, N_PAD - N), (0, 0)))
    batch_pad = jnp.concatenate([batch, jnp.full((N_PAD - N,), G, jnp.int32)])
    batch3 = batch_pad.reshape(NBLK, 1, BLK)
    zeros1 = jnp.zeros((N_PAD,), f32)
    zeros2 = jnp.zeros((RPT, H), f32)

    degT = _deg_call()(col2d, zeros1)
    deg2 = degT.T

    u1, counts = _first_call(x_pad, W1, deg2, batch3)
    p = _agg_call()(u1, row2d, col4d, zeros2)
    u2, pool1 = _layer_call(p, u1, deg2, b1.reshape(1, H), W2, batch3)
    p = _agg_call()(u2, row2d, col4d, zeros2)
    u3, pool2 = _layer_call(p, u2, deg2, b2.reshape(1, H), W3, batch3)
    p = _agg_call()(u3, row2d, col4d, zeros2)
    pool3 = _last_call(p, u3, deg2, b3.reshape(1, H), batch3)

    lin2_Wp = jnp.pad(lin2_W, ((0, 0), (0, 128 - OUT)))
    lin2_bp = jnp.pad(lin2_b, (0, 128 - OUT)).reshape(1, 128)
    hout, ls = _head_call(pool1, pool2, pool3, counts,
                          lin1_W, lin1_b.reshape(1, 3 * H), lin2_Wp, lin2_bp)
    return (hout[:, :OUT], ls[:, :OUT])


# speedup vs baseline: 30.3260x; 1.2971x over previous
import functools

import jax
import jax.numpy as jnp
from jax import lax
from jax.experimental import pallas as pl
from jax.experimental.pallas import tpu as pltpu
from jax.experimental.pallas import tpu_sc as plsc

N = 10000
E = 320000
D = 128
H = 128
G = 128
OUT = 16

N_PAD = 10240
BLK = 1024
NBLK = N_PAD // BLK

NC = 2
NS = 16
NW = NC * NS
EPW = E // NW
CH = 80
NCH = EPW // CH
RPT = N_PAD // NS

NBUF = 3
NGRP = NCH // NBUF
NTAIL = NCH - NGRP * NBUF


@functools.lru_cache(maxsize=1)
def _sc_mesh():
    return plsc.VectorSubcoreMesh(core_axis_name="c", subcore_axis_name="s",
                                  num_cores=NC, num_subcores=NS)


def _deg_body(col2d_hbm, zeros1_hbm, deg_hbm, acc_sh, colslab, onesv, sem):
    del sem
    c = lax.axis_index("c")
    s = lax.axis_index("s")
    wid = c * NS + s
    pltpu.sync_copy(zeros1_hbm.at[pl.ds(s * RPT, RPT)], acc_sh.at[pl.ds(s * RPT, RPT)])
    pltpu.sync_copy(col2d_hbm.at[wid], colslab)
    for k in range(CH // 16):
        onesv[pl.ds(k * 16, 16)] = jnp.ones((16,), jnp.float32)
    plsc.subcore_barrier()

    def chunk(j, carry):
        pltpu.sync_copy(onesv, acc_sh.at[colslab.at[j]], add=True)
        return carry

    lax.fori_loop(0, NCH, chunk, 0)
    plsc.subcore_barrier()
    pltpu.sync_copy(acc_sh.at[pl.ds(s * RPT, RPT)], deg_hbm.at[c, pl.ds(s * RPT, RPT)])


@functools.lru_cache(maxsize=1)
def _deg_call():
    return pl.kernel(
        _deg_body,
        out_type=jax.ShapeDtypeStruct((NC, N_PAD), jnp.float32),
        mesh=_sc_mesh(),
        scratch_types=[
            pltpu.VMEM_SHARED((N_PAD,), jnp.float32),
            pltpu.VMEM((NCH, CH), jnp.int32),
            pltpu.VMEM((CH,), jnp.float32),
            pltpu.SemaphoreType.DMA,
        ],
    )


def _agg_body(u_hbm, row2d_hbm, col4d_hbm, zeros2_hbm, p_hbm,
              acc_sh, rowslab, rowsv, coltmp, gsems, csems, ssems):
    c = lax.axis_index("c")
    s = lax.axis_index("s")
    wid = c * NS + s
    pltpu.sync_copy(zeros2_hbm, acc_sh.at[pl.ds(s * RPT, RPT)])
    pltpu.sync_copy(row2d_hbm.at[wid], rowslab)
    plsc.subcore_barrier()

    def fire(j, b):
        pltpu.async_copy(u_hbm.at[rowslab.at[j]], rowsv.at[b], gsems.at[b])
        pltpu.async_copy(col4d_hbm.at[wid, j], coltmp.at[b], csems.at[b])

    def wait_gather(j, b):
        pltpu.make_async_copy(u_hbm.at[rowslab.at[j]], rowsv.at[b],
                              gsems.at[b]).wait()
        pltpu.make_async_copy(col4d_hbm.at[wid, j], coltmp.at[b],
                              csems.at[b]).wait()

    def fire_scatter(b):
        pltpu.async_copy(rowsv.at[b], acc_sh.at[coltmp.at[b, 0]],
                         ssems.at[b], add=True)

    def wait_scatter(b):
        pltpu.make_async_copy(rowsv.at[b], acc_sh.at[coltmp.at[b, 0]],
                              ssems.at[b]).wait()

    fire(0, 0)
    fire(1, 1)

    def step(j, b):
        wait_gather(j, b)
        fire_scatter(b)
        b2 = (b + 2) % NBUF

        @pl.when(j >= 1)
        def _():
            wait_scatter(b2)

        fire(j + 2, b2)

    def outer(g, carry):
        for b in range(NBUF):
            step(g * NBUF + b, b)
        return carry

    lax.fori_loop(0, (NCH - NTAIL) // NBUF, outer, 0)
    for j in range(NCH - NTAIL, NCH):
        b = j % NBUF
        wait_gather(j, b)
        wait_scatter((b + 2) % NBUF)
        fire_scatter(b)
    wait_scatter((NCH - 1) % NBUF)
    plsc.subcore_barrier()
    pltpu.sync_copy(acc_sh.at[pl.ds(s * RPT, RPT)], p_hbm.at[c, pl.ds(s * RPT, RPT)])


@functools.lru_cache(maxsize=1)
def _agg_call():
    return pl.kernel(
        _agg_body,
        out_type=jax.ShapeDtypeStruct((NC, N_PAD, H), jnp.float32),
        mesh=_sc_mesh(),
        scratch_types=[
            pltpu.VMEM_SHARED((N_PAD, H), jnp.float32),
            pltpu.VMEM((NCH, CH), jnp.int32),
            pltpu.VMEM((NBUF, CH, H), jnp.float32),
            pltpu.VMEM((NBUF, 1, CH), jnp.int32),
            pltpu.SemaphoreType.DMA((NBUF,)),
            pltpu.SemaphoreType.DMA((NBUF,)),
            pltpu.SemaphoreType.DMA((NBUF,)),
        ],
    )


def _dinv_from(deg_ref):
    deg = 1.0 + jnp.sum(deg_ref[...], axis=1, keepdims=True)
    return lax.rsqrt(deg)


def _onehot(b_ref):
    bb = b_ref[0]
    gid = lax.broadcasted_iota(jnp.int32, (G, BLK), 0)
    return (jnp.broadcast_to(bb, (G, BLK)) == gid).astype(jnp.float32)


def _first_body(x_ref, w_ref, deg_ref, b_ref, u_ref, cnt_ref):
    i = pl.program_id(0)
    dinv = _dinv_from(deg_ref)
    xw = jnp.dot(x_ref[...], w_ref[...], preferred_element_type=jnp.float32)
    u_ref[...] = dinv * xw
    oh = _onehot(b_ref)

    @pl.when(i == 0)
    def _():
        cnt_ref[...] = jnp.zeros_like(cnt_ref)

    cnt_ref[...] += jnp.dot(oh, jnp.ones((BLK, G), jnp.float32),
                            preferred_element_type=jnp.float32)


def _first_call(x, W, deg2, batch3):
    return pl.pallas_call(
        _first_body,
        grid=(NBLK,),
        in_specs=[
            pl.BlockSpec((BLK, D), lambda i: (i, 0)),
            pl.BlockSpec((D, H), lambda i: (0, 0)),
            pl.BlockSpec((BLK, 2), lambda i: (i, 0)),
            pl.BlockSpec((1, 1, BLK), lambda i: (i, 0, 0)),
        ],
        out_specs=[
            pl.BlockSpec((BLK, H), lambda i: (i, 0)),
            pl.BlockSpec((G, G), lambda i: (0, 0)),
        ],
        out_shape=[
            jax.ShapeDtypeStruct((N_PAD, H), jnp.float32),
            jax.ShapeDtypeStruct((G, G), jnp.float32),
        ],
    )(x, W, deg2, batch3)


def _layer_body(p_ref, u_ref, deg_ref, bias_ref, w_ref, b_ref,
                unext_ref, pool_ref):
    i = pl.program_id(0)
    dinv = _dinv_from(deg_ref)
    agg = p_ref[0] + p_ref[1]
    h = dinv * (agg + u_ref[...]) + bias_ref[...]
    unext_ref[...] = dinv * jnp.dot(h, w_ref[...], preferred_element_type=jnp.float32)
    oh = _onehot(b_ref)

    @pl.when(i == 0)
    def _():
        pool_ref[...] = jnp.zeros_like(pool_ref)

    pool_ref[...] += jnp.dot(oh, h, preferred_element_type=jnp.float32)


def _layer_call(p, u, deg2, bias, W, batch3):
    return pl.pallas_call(
        _layer_body,
        grid=(NBLK,),
        in_specs=[
            pl.BlockSpec((NC, BLK, H), lambda i: (0, i, 0)),
            pl.BlockSpec((BLK, H), lambda i: (i, 0)),
            pl.BlockSpec((BLK, 2), lambda i: (i, 0)),
            pl.BlockSpec((1, H), lambda i: (0, 0)),
            pl.BlockSpec((H, H), lambda i: (0, 0)),
            pl.BlockSpec((1, 1, BLK), lambda i: (i, 0, 0)),
        ],
        out_specs=[
            pl.BlockSpec((BLK, H), lambda i: (i, 0)),
            pl.BlockSpec((G, H), lambda i: (0, 0)),
        ],
        out_shape=[
            jax.ShapeDtypeStruct((N_PAD, H), jnp.float32),
            jax.ShapeDtypeStruct((G, H), jnp.float32),
        ],
    )(p, u, deg2, bias, W, batch3)


def _last_body(p_ref, u_ref, deg_ref, bias_ref, b_ref, pool_ref):
    i = pl.program_id(0)
    dinv = _dinv_from(deg_ref)
    agg = p_ref[0] + p_ref[1]
    h = dinv * (agg + u_ref[...]) + bias_ref[...]
    oh = _onehot(b_ref)

    @pl.when(i == 0)
    def _():
        pool_ref[...] = jnp.zeros_like(pool_ref)

    pool_ref[...] += jnp.dot(oh, h, preferred_element_type=jnp.float32)


def _last_call(p, u, deg2, bias, batch3):
    return pl.pallas_call(
        _last_body,
        grid=(NBLK,),
        in_specs=[
            pl.BlockSpec((NC, BLK, H), lambda i: (0, i, 0)),
            pl.BlockSpec((BLK, H), lambda i: (i, 0)),
            pl.BlockSpec((BLK, 2), lambda i: (i, 0)),
            pl.BlockSpec((1, H), lambda i: (0, 0)),
            pl.BlockSpec((1, 1, BLK), lambda i: (i, 0, 0)),
        ],
        out_specs=pl.BlockSpec((G, H), lambda i: (0, 0)),
        out_shape=jax.ShapeDtypeStruct((G, H), jnp.float32),
    )(p, u, deg2, bias, batch3)


def _head_body(p1_ref, p2_ref, p3_ref, cnt_ref, w1_ref, b1_ref, w2_ref, b2_ref,
               h_ref, ls_ref):
    cnt = jnp.maximum(cnt_ref[...][:, 0:1], 1.0)
    poolcat = jnp.concatenate([p1_ref[...], p2_ref[...], p3_ref[...]], axis=1) / cnt
    t = jnp.dot(poolcat, w1_ref[...], preferred_element_type=jnp.float32) + b1_ref[...]
    t = jnp.maximum(t, 0.0)
    o = jnp.dot(t, w2_ref[...], preferred_element_type=jnp.float32) + b2_ref[...]
    h_ref[...] = o
    colid = lax.broadcasted_iota(jnp.int32, (G, 128), 1)
    valid = colid < OUT
    om = jnp.where(valid, o, -1e30)
    m = jnp.max(om, axis=1, keepdims=True)
    ssum = jnp.sum(jnp.where(valid, jnp.exp(om - m), 0.0), axis=1, keepdims=True)
    ls_ref[...] = om - m - jnp.log(ssum)


def _head_call(pool1, pool2, pool3, counts, lin1_W, lin1_b, lin2_Wp, lin2_bp):
    HH = 3 * H
    return pl.pallas_call(
        _head_body,
        in_specs=[
            pl.BlockSpec((G, H), lambda: (0, 0)),
            pl.BlockSpec((G, H), lambda: (0, 0)),
            pl.BlockSpec((G, H), lambda: (0, 0)),
            pl.BlockSpec((G, G), lambda: (0, 0)),
            pl.BlockSpec((HH, HH), lambda: (0, 0)),
            pl.BlockSpec((1, HH), lambda: (0, 0)),
            pl.BlockSpec((HH, 128), lambda: (0, 0)),
            pl.BlockSpec((1, 128), lambda: (0, 0)),
        ],
        out_specs=[
            pl.BlockSpec((G, 128), lambda: (0, 0)),
            pl.BlockSpec((G, 128), lambda: (0, 0)),
        ],
        out_shape=[
            jax.ShapeDtypeStruct((G, 128), jnp.float32),
            jax.ShapeDtypeStruct((G, 128), jnp.float32),
        ],
    )(pool1, pool2, pool3, counts, lin1_W, lin1_b, lin2_Wp, lin2_bp)


def kernel(x, edge_index, batch, W1, b1, W2, b2, W3, b3,
           lin1_W, lin1_b, lin2_W, lin2_b):
    f32 = jnp.float32
    row2d = edge_index[0].reshape(NW, NCH, CH)
    col2d = edge_index[1].reshape(NW, NCH, CH)
    col4d = edge_index[1].reshape(NW, NCH, 1, CH)
    x_pad = jnp.pad(x, ((0, N_PAD - N), (0, 0)))
    batch_pad = jnp.concatenate([batch, jnp.full((N_PAD - N,), G, jnp.int32)])
    batch3 = batch_pad.reshape(NBLK, 1, BLK)
    zeros1 = jnp.zeros((N_PAD,), f32)
    zeros2 = jnp.zeros((RPT, H), f32)

    degT = _deg_call()(col2d, zeros1)
    deg2 = degT.T

    u1, counts = _first_call(x_pad, W1, deg2, batch3)
    p = _agg_call()(u1, row2d, col4d, zeros2)
    u2, pool1 = _layer_call(p, u1, deg2, b1.reshape(1, H), W2, batch3)
    p = _agg_call()(u2, row2d, col4d, zeros2)
    u3, pool2 = _layer_call(p, u2, deg2, b2.reshape(1, H), W3, batch3)
    p = _agg_call()(u3, row2d, col4d, zeros2)
    pool3 = _last_call(p, u3, deg2, b3.reshape(1, H), batch3)

    lin2_Wp = jnp.pad(lin2_W, ((0, 0), (0, 128 - OUT)))
    lin2_bp = jnp.pad(lin2_b, (0, 128 - OUT)).reshape(1, 128)
    hout, ls = _head_call(pool1, pool2, pool3, counts,
                          lin1_W, lin1_b.reshape(1, 3 * H), lin2_Wp, lin2_bp)
    return (hout[:, :OUT], ls[:, :OUT])


# speedup vs baseline: 30.8980x; 1.0189x over previous
import functools

import jax
import jax.numpy as jnp
from jax import lax
from jax.experimental import pallas as pl
from jax.experimental.pallas import tpu as pltpu
from jax.experimental.pallas import tpu_sc as plsc

N = 10000
E = 320000
D = 128
H = 128
G = 128
OUT = 16

N_PAD = 10240
BLK = 1024
NBLK = N_PAD // BLK

NC = 2
NS = 16
NW = NC * NS
EPW = E // NW
CH = 80
NCH = EPW // CH
RPT = N_PAD // NS

NBUF = 3
NGRP = NCH // NBUF
NTAIL = NCH - NGRP * NBUF


@functools.lru_cache(maxsize=1)
def _sc_mesh():
    return plsc.VectorSubcoreMesh(core_axis_name="c", subcore_axis_name="s",
                                  num_cores=NC, num_subcores=NS)


def _deg_body(col2d_hbm, zeros1_hbm, deg_hbm, acc_sh, colslab, onesv, sem):
    del sem
    c = lax.axis_index("c")
    s = lax.axis_index("s")
    wid = c * NS + s
    pltpu.sync_copy(zeros1_hbm.at[pl.ds(s * RPT, RPT)], acc_sh.at[pl.ds(s * RPT, RPT)])
    pltpu.sync_copy(col2d_hbm.at[wid], colslab)
    for k in range(CH // 16):
        onesv[pl.ds(k * 16, 16)] = jnp.ones((16,), jnp.float32)
    plsc.subcore_barrier()

    def chunk(j, carry):
        pltpu.sync_copy(onesv, acc_sh.at[colslab.at[j]], add=True)
        return carry

    lax.fori_loop(0, NCH, chunk, 0)
    plsc.subcore_barrier()
    pltpu.sync_copy(acc_sh.at[pl.ds(s * RPT, RPT)], deg_hbm.at[c, pl.ds(s * RPT, RPT)])


@functools.lru_cache(maxsize=1)
def _deg_call():
    return pl.kernel(
        _deg_body,
        out_type=jax.ShapeDtypeStruct((NC, N_PAD), jnp.float32),
        mesh=_sc_mesh(),
        scratch_types=[
            pltpu.VMEM_SHARED((N_PAD,), jnp.float32),
            pltpu.VMEM((NCH, CH), jnp.int32),
            pltpu.VMEM((CH,), jnp.float32),
            pltpu.SemaphoreType.DMA,
        ],
    )


def _agg_body(u_hbm, row2d_hbm, col4d_hbm, zeros2_hbm, p_hbm,
              acc_sh, rowslab, rowsv, coltmp, gsems, csems, ssems, zsem):
    c = lax.axis_index("c")
    s = lax.axis_index("s")
    wid = c * NS + s
    dz = pltpu.async_copy(zeros2_hbm, acc_sh.at[pl.ds(s * RPT, RPT)], zsem)
    pltpu.sync_copy(row2d_hbm.at[wid], rowslab)

    def fire(j, b):
        pltpu.async_copy(u_hbm.at[rowslab.at[j]], rowsv.at[b], gsems.at[b])
        pltpu.async_copy(col4d_hbm.at[wid, j], coltmp.at[b], csems.at[b])

    def wait_gather(j, b):
        pltpu.make_async_copy(u_hbm.at[rowslab.at[j]], rowsv.at[b],
                              gsems.at[b]).wait()
        pltpu.make_async_copy(col4d_hbm.at[wid, j], coltmp.at[b],
                              csems.at[b]).wait()

    def fire_scatter(b):
        pltpu.async_copy(rowsv.at[b], acc_sh.at[coltmp.at[b, 0]],
                         ssems.at[b], add=True)

    def wait_scatter(b):
        pltpu.make_async_copy(rowsv.at[b], acc_sh.at[coltmp.at[b, 0]],
                              ssems.at[b]).wait()

    fire(0, 0)
    fire(1, 1)
    dz.wait()
    plsc.subcore_barrier()

    def step(j, b):
        wait_gather(j, b)
        fire_scatter(b)
        b2 = (b + 2) % NBUF

        @pl.when(j >= 1)
        def _():
            wait_scatter(b2)

        fire(j + 2, b2)

    def outer(g, carry):
        for b in range(NBUF):
            step(g * NBUF + b, b)
        return carry

    lax.fori_loop(0, (NCH - NTAIL) // NBUF, outer, 0)
    for j in range(NCH - NTAIL, NCH):
        b = j % NBUF
        wait_gather(j, b)
        wait_scatter((b + 2) % NBUF)
        fire_scatter(b)
    wait_scatter((NCH - 1) % NBUF)
    plsc.subcore_barrier()
    pltpu.sync_copy(acc_sh.at[pl.ds(s * RPT, RPT)], p_hbm.at[c, pl.ds(s * RPT, RPT)])


@functools.lru_cache(maxsize=1)
def _agg_call():
    return pl.kernel(
        _agg_body,
        out_type=jax.ShapeDtypeStruct((NC, N_PAD, H), jnp.float32),
        mesh=_sc_mesh(),
        scratch_types=[
            pltpu.VMEM_SHARED((N_PAD, H), jnp.float32),
            pltpu.VMEM((NCH, CH), jnp.int32),
            pltpu.VMEM((NBUF, CH, H), jnp.float32),
            pltpu.VMEM((NBUF, 1, CH), jnp.int32),
            pltpu.SemaphoreType.DMA((NBUF,)),
            pltpu.SemaphoreType.DMA((NBUF,)),
            pltpu.SemaphoreType.DMA((NBUF,)),
            pltpu.SemaphoreType.DMA,
        ],
    )


def _dinv_from(deg_ref):
    deg = 1.0 + jnp.sum(deg_ref[...], axis=1, keepdims=True)
    return lax.rsqrt(deg)


def _onehot(b_ref):
    bb = b_ref[0]
    gid = lax.broadcasted_iota(jnp.int32, (G, BLK), 0)
    return (jnp.broadcast_to(bb, (G, BLK)) == gid).astype(jnp.float32)


def _first_body(x_ref, w_ref, deg_ref, b_ref, u_ref, cnt_ref):
    i = pl.program_id(0)
    dinv = _dinv_from(deg_ref)
    xw = jnp.dot(x_ref[...], w_ref[...], preferred_element_type=jnp.float32)
    u_ref[...] = dinv * xw
    oh = _onehot(b_ref)

    @pl.when(i == 0)
    def _():
        cnt_ref[...] = jnp.zeros_like(cnt_ref)

    cnt_ref[...] += jnp.dot(oh, jnp.ones((BLK, G), jnp.float32),
                            preferred_element_type=jnp.float32)


def _first_call(x, W, deg2, batch3):
    return pl.pallas_call(
        _first_body,
        grid=(NBLK,),
        in_specs=[
            pl.BlockSpec((BLK, D), lambda i: (i, 0)),
            pl.BlockSpec((D, H), lambda i: (0, 0)),
            pl.BlockSpec((BLK, 2), lambda i: (i, 0)),
            pl.BlockSpec((1, 1, BLK), lambda i: (i, 0, 0)),
        ],
        out_specs=[
            pl.BlockSpec((BLK, H), lambda i: (i, 0)),
            pl.BlockSpec((G, G), lambda i: (0, 0)),
        ],
        out_shape=[
            jax.ShapeDtypeStruct((N_PAD, H), jnp.float32),
            jax.ShapeDtypeStruct((G, G), jnp.float32),
        ],
    )(x, W, deg2, batch3)


def _layer_body(p_ref, u_ref, deg_ref, bias_ref, w_ref, b_ref,
                unext_ref, pool_ref):
    i = pl.program_id(0)
    dinv = _dinv_from(deg_ref)
    agg = p_ref[0] + p_ref[1]
    h = dinv * (agg + u_ref[...]) + bias_ref[...]
    unext_ref[...] = dinv * jnp.dot(h, w_ref[...], preferred_element_type=jnp.float32)
    oh = _onehot(b_ref)

    @pl.when(i == 0)
    def _():
        pool_ref[...] = jnp.zeros_like(pool_ref)

    pool_ref[...] += jnp.dot(oh, h, preferred_element_type=jnp.float32)


def _layer_call(p, u, deg2, bias, W, batch3):
    return pl.pallas_call(
        _layer_body,
        grid=(NBLK,),
        in_specs=[
            pl.BlockSpec((NC, BLK, H), lambda i: (0, i, 0)),
            pl.BlockSpec((BLK, H), lambda i: (i, 0)),
            pl.BlockSpec((BLK, 2), lambda i: (i, 0)),
            pl.BlockSpec((1, H), lambda i: (0, 0)),
            pl.BlockSpec((H, H), lambda i: (0, 0)),
            pl.BlockSpec((1, 1, BLK), lambda i: (i, 0, 0)),
        ],
        out_specs=[
            pl.BlockSpec((BLK, H), lambda i: (i, 0)),
            pl.BlockSpec((G, H), lambda i: (0, 0)),
        ],
        out_shape=[
            jax.ShapeDtypeStruct((N_PAD, H), jnp.float32),
            jax.ShapeDtypeStruct((G, H), jnp.float32),
        ],
    )(p, u, deg2, bias, W, batch3)


def _last_body(p_ref, u_ref, deg_ref, bias_ref, b_ref,
               pool1_ref, pool2_ref, cnt_ref, w1_ref, b1_ref, w2_ref, b2_ref,
               h_ref, ls_ref, pool_scr):
    i = pl.program_id(0)
    dinv = _dinv_from(deg_ref)
    agg = p_ref[0] + p_ref[1]
    h = dinv * (agg + u_ref[...]) + bias_ref[...]
    oh = _onehot(b_ref)

    @pl.when(i == 0)
    def _():
        pool_scr[...] = jnp.zeros_like(pool_scr)

    pool_scr[...] += jnp.dot(oh, h, preferred_element_type=jnp.float32)

    @pl.when(i == NBLK - 1)
    def _():
        cnt = jnp.maximum(cnt_ref[...][:, 0:1], 1.0)
        poolcat = jnp.concatenate(
            [pool1_ref[...], pool2_ref[...], pool_scr[...]], axis=1) / cnt
        t = jnp.dot(poolcat, w1_ref[...],
                    preferred_element_type=jnp.float32) + b1_ref[...]
        t = jnp.maximum(t, 0.0)
        o = jnp.dot(t, w2_ref[...],
                    preferred_element_type=jnp.float32) + b2_ref[...]
        h_ref[...] = o
        colid = lax.broadcasted_iota(jnp.int32, (G, 128), 1)
        valid = colid < OUT
        om = jnp.where(valid, o, -1e30)
        m = jnp.max(om, axis=1, keepdims=True)
        ssum = jnp.sum(jnp.where(valid, jnp.exp(om - m), 0.0),
                       axis=1, keepdims=True)
        ls_ref[...] = om - m - jnp.log(ssum)


def _last_call(p, u, deg2, bias, batch3, pool1, pool2, counts,
               lin1_W, lin1_b, lin2_Wp, lin2_bp):
    HH = 3 * H
    return pl.pallas_call(
        _last_body,
        grid=(NBLK,),
        in_specs=[
            pl.BlockSpec((NC, BLK, H), lambda i: (0, i, 0)),
            pl.BlockSpec((BLK, H), lambda i: (i, 0)),
            pl.BlockSpec((BLK, 2), lambda i: (i, 0)),
            pl.BlockSpec((1, H), lambda i: (0, 0)),
            pl.BlockSpec((1, 1, BLK), lambda i: (i, 0, 0)),
            pl.BlockSpec((G, H), lambda i: (0, 0)),
            pl.BlockSpec((G, H), lambda i: (0, 0)),
            pl.BlockSpec((G, G), lambda i: (0, 0)),
            pl.BlockSpec((HH, HH), lambda i: (0, 0)),
            pl.BlockSpec((1, HH), lambda i: (0, 0)),
            pl.BlockSpec((HH, 128), lambda i: (0, 0)),
            pl.BlockSpec((1, 128), lambda i: (0, 0)),
        ],
        out_specs=[
            pl.BlockSpec((G, 128), lambda i: (0, 0)),
            pl.BlockSpec((G, 128), lambda i: (0, 0)),
        ],
        out_shape=[
            jax.ShapeDtypeStruct((G, 128), jnp.float32),
            jax.ShapeDtypeStruct((G, 128), jnp.float32),
        ],
        scratch_shapes=[pltpu.VMEM((G, H), jnp.float32)],
    )(p, u, deg2, bias, batch3, pool1, pool2, counts,
      lin1_W, lin1_b, lin2_Wp, lin2_bp)


def kernel(x, edge_index, batch, W1, b1, W2, b2, W3, b3,
           lin1_W, lin1_b, lin2_W, lin2_b):
    f32 = jnp.float32
    row2d = edge_index[0].reshape(NW, NCH, CH)
    col2d = edge_index[1].reshape(NW, NCH, CH)
    col4d = edge_index[1].reshape(NW, NCH, 1, CH)
    x_pad = jnp.pad(x, ((0, N_PAD - N), (0, 0)))
    batch_pad = jnp.concatenate([batch, jnp.full((N_PAD - N,), G, jnp.int32)])
    batch3 = batch_pad.reshape(NBLK, 1, BLK)
    zeros1 = jnp.zeros((N_PAD,), f32)
    zeros2 = jnp.zeros((RPT, H), f32)

    degT = _deg_call()(col2d, zeros1)
    deg2 = degT.T

    u1, counts = _first_call(x_pad, W1, deg2, batch3)
    p = _agg_call()(u1, row2d, col4d, zeros2)
    u2, pool1 = _layer_call(p, u1, deg2, b1.reshape(1, H), W2, batch3)
    p = _agg_call()(u2, row2d, col4d, zeros2)
    u3, pool2 = _layer_call(p, u2, deg2, b2.reshape(1, H), W3, batch3)
    p = _agg_call()(u3, row2d, col4d, zeros2)
    lin2_Wp = jnp.pad(lin2_W, ((0, 0), (0, 128 - OUT)))
    lin2_bp = jnp.pad(lin2_b, (0, 128 - OUT)).reshape(1, 128)
    hout, ls = _last_call(p, u3, deg2, b3.reshape(1, H), batch3,
                          pool1, pool2, counts,
                          lin1_W, lin1_b.reshape(1, 3 * H), lin2_Wp, lin2_bp)
    return (hout[:, :OUT], ls[:, :OUT])


# speedup vs baseline: 30.9419x; 1.0014x over previous
import functools

import jax
import jax.numpy as jnp
from jax import lax
from jax.experimental import pallas as pl
from jax.experimental.pallas import tpu as pltpu
from jax.experimental.pallas import tpu_sc as plsc

N = 10000
E = 320000
D = 128
H = 128
G = 128
OUT = 16

N_PAD = 10240
BLK = 1024
NBLK = N_PAD // BLK

NC = 2
NS = 16
NW = NC * NS
EPW = E // NW
CH = 80
NCH = EPW // CH
RPT = N_PAD // NS

NBUF = 3
NGRP = NCH // NBUF
NTAIL = NCH - NGRP * NBUF


@functools.lru_cache(maxsize=1)
def _sc_mesh():
    return plsc.VectorSubcoreMesh(core_axis_name="c", subcore_axis_name="s",
                                  num_cores=NC, num_subcores=NS)


MAXQ = 8


def _deg_body(col2d_hbm, zeros1_hbm, deg_hbm, acc_sh, colslab, onesv, sem):
    c = lax.axis_index("c")
    s = lax.axis_index("s")
    wid = c * NS + s
    pltpu.sync_copy(zeros1_hbm.at[pl.ds(s * RPT, RPT)], acc_sh.at[pl.ds(s * RPT, RPT)])
    pltpu.sync_copy(col2d_hbm.at[wid], colslab)
    for k in range(CH // 16):
        onesv[pl.ds(k * 16, 16)] = jnp.ones((16,), jnp.float32)
    plsc.subcore_barrier()

    def chunk(j, carry):
        @pl.when(j >= MAXQ)
        def _():
            pltpu.make_async_copy(onesv, acc_sh.at[colslab.at[j - MAXQ]],
                                  sem).wait()

        pltpu.async_copy(onesv, acc_sh.at[colslab.at[j]], sem, add=True)
        return carry

    lax.fori_loop(0, NCH, chunk, 0)
    for k in range(NCH - MAXQ, NCH):
        pltpu.make_async_copy(onesv, acc_sh.at[colslab.at[k]], sem).wait()
    plsc.subcore_barrier()
    pltpu.sync_copy(acc_sh.at[pl.ds(s * RPT, RPT)], deg_hbm.at[c, pl.ds(s * RPT, RPT)])


@functools.lru_cache(maxsize=1)
def _deg_call():
    return pl.kernel(
        _deg_body,
        out_type=jax.ShapeDtypeStruct((NC, N_PAD), jnp.float32),
        mesh=_sc_mesh(),
        scratch_types=[
            pltpu.VMEM_SHARED((N_PAD,), jnp.float32),
            pltpu.VMEM((NCH, CH), jnp.int32),
            pltpu.VMEM((CH,), jnp.float32),
            pltpu.SemaphoreType.DMA,
        ],
    )


def _agg_body(u_hbm, row2d_hbm, col4d_hbm, zeros2_hbm, p_hbm,
              acc_sh, rowslab, rowsv, coltmp, gsems, csems, ssems, zsem):
    c = lax.axis_index("c")
    s = lax.axis_index("s")
    wid = c * NS + s
    dz = pltpu.async_copy(zeros2_hbm, acc_sh.at[pl.ds(s * RPT, RPT)], zsem)
    pltpu.sync_copy(row2d_hbm.at[wid], rowslab)

    def fire(j, b):
        pltpu.async_copy(u_hbm.at[rowslab.at[j]], rowsv.at[b], gsems.at[b])
        pltpu.async_copy(col4d_hbm.at[wid, j], coltmp.at[b], csems.at[b])

    def wait_gather(j, b):
        pltpu.make_async_copy(u_hbm.at[rowslab.at[j]], rowsv.at[b],
                              gsems.at[b]).wait()
        pltpu.make_async_copy(col4d_hbm.at[wid, j], coltmp.at[b],
                              csems.at[b]).wait()

    def fire_scatter(b):
        pltpu.async_copy(rowsv.at[b], acc_sh.at[coltmp.at[b, 0]],
                         ssems.at[b], add=True)

    def wait_scatter(b):
        pltpu.make_async_copy(rowsv.at[b], acc_sh.at[coltmp.at[b, 0]],
                              ssems.at[b]).wait()

    fire(0, 0)
    fire(1, 1)
    dz.wait()
    plsc.subcore_barrier()

    def step(j, b):
        wait_gather(j, b)
        fire_scatter(b)
        b2 = (b + 2) % NBUF

        @pl.when(j >= 1)
        def _():
            wait_scatter(b2)

        fire(j + 2, b2)

    def outer(g, carry):
        for b in range(NBUF):
            step(g * NBUF + b, b)
        return carry

    lax.fori_loop(0, (NCH - NTAIL) // NBUF, outer, 0)
    for j in range(NCH - NTAIL, NCH):
        b = j % NBUF
        wait_gather(j, b)
        wait_scatter((b + 2) % NBUF)
        fire_scatter(b)
    wait_scatter((NCH - 1) % NBUF)
    plsc.subcore_barrier()
    pltpu.sync_copy(acc_sh.at[pl.ds(s * RPT, RPT)], p_hbm.at[c, pl.ds(s * RPT, RPT)])


@functools.lru_cache(maxsize=1)
def _agg_call():
    return pl.kernel(
        _agg_body,
        out_type=jax.ShapeDtypeStruct((NC, N_PAD, H), jnp.float32),
        mesh=_sc_mesh(),
        scratch_types=[
            pltpu.VMEM_SHARED((N_PAD, H), jnp.float32),
            pltpu.VMEM((NCH, CH), jnp.int32),
            pltpu.VMEM((NBUF, CH, H), jnp.float32),
            pltpu.VMEM((NBUF, 1, CH), jnp.int32),
            pltpu.SemaphoreType.DMA((NBUF,)),
            pltpu.SemaphoreType.DMA((NBUF,)),
            pltpu.SemaphoreType.DMA((NBUF,)),
            pltpu.SemaphoreType.DMA,
        ],
    )


def _dinv_from(deg_ref):
    deg = 1.0 + jnp.sum(deg_ref[...], axis=1, keepdims=True)
    return lax.rsqrt(deg)


def _onehot(b_ref):
    bb = b_ref[0]
    gid = lax.broadcasted_iota(jnp.int32, (G, BLK), 0)
    return (jnp.broadcast_to(bb, (G, BLK)) == gid).astype(jnp.float32)


def _first_body(x_ref, w_ref, deg_ref, b_ref, u_ref, cnt_ref):
    i = pl.program_id(0)
    dinv = _dinv_from(deg_ref)
    xw = jnp.dot(x_ref[...], w_ref[...], preferred_element_type=jnp.float32)
    u_ref[...] = dinv * xw
    oh = _onehot(b_ref)

    @pl.when(i == 0)
    def _():
        cnt_ref[...] = jnp.zeros_like(cnt_ref)

    cnt_ref[...] += jnp.dot(oh, jnp.ones((BLK, G), jnp.float32),
                            preferred_element_type=jnp.float32)


def _first_call(x, W, deg2, batch3):
    return pl.pallas_call(
        _first_body,
        grid=(NBLK,),
        in_specs=[
            pl.BlockSpec((BLK, D), lambda i: (i, 0)),
            pl.BlockSpec((D, H), lambda i: (0, 0)),
            pl.BlockSpec((BLK, 2), lambda i: (i, 0)),
            pl.BlockSpec((1, 1, BLK), lambda i: (i, 0, 0)),
        ],
        out_specs=[
            pl.BlockSpec((BLK, H), lambda i: (i, 0)),
            pl.BlockSpec((G, G), lambda i: (0, 0)),
        ],
        out_shape=[
            jax.ShapeDtypeStruct((N_PAD, H), jnp.float32),
            jax.ShapeDtypeStruct((G, G), jnp.float32),
        ],
    )(x, W, deg2, batch3)


def _layer_body(p_ref, u_ref, deg_ref, bias_ref, w_ref, b_ref,
                unext_ref, pool_ref):
    i = pl.program_id(0)
    dinv = _dinv_from(deg_ref)
    agg = p_ref[0] + p_ref[1]
    h = dinv * (agg + u_ref[...]) + bias_ref[...]
    unext_ref[...] = dinv * jnp.dot(h, w_ref[...], preferred_element_type=jnp.float32)
    oh = _onehot(b_ref)

    @pl.when(i == 0)
    def _():
        pool_ref[...] = jnp.zeros_like(pool_ref)

    pool_ref[...] += jnp.dot(oh, h, preferred_element_type=jnp.float32)


def _layer_call(p, u, deg2, bias, W, batch3):
    return pl.pallas_call(
        _layer_body,
        grid=(NBLK,),
        in_specs=[
            pl.BlockSpec((NC, BLK, H), lambda i: (0, i, 0)),
            pl.BlockSpec((BLK, H), lambda i: (i, 0)),
            pl.BlockSpec((BLK, 2), lambda i: (i, 0)),
            pl.BlockSpec((1, H), lambda i: (0, 0)),
            pl.BlockSpec((H, H), lambda i: (0, 0)),
            pl.BlockSpec((1, 1, BLK), lambda i: (i, 0, 0)),
        ],
        out_specs=[
            pl.BlockSpec((BLK, H), lambda i: (i, 0)),
            pl.BlockSpec((G, H), lambda i: (0, 0)),
        ],
        out_shape=[
            jax.ShapeDtypeStruct((N_PAD, H), jnp.float32),
            jax.ShapeDtypeStruct((G, H), jnp.float32),
        ],
    )(p, u, deg2, bias, W, batch3)


def _last_body(p_ref, u_ref, deg_ref, bias_ref, b_ref,
               pool1_ref, pool2_ref, cnt_ref, w1_ref, b1_ref, w2_ref, b2_ref,
               h_ref, ls_ref, pool_scr):
    i = pl.program_id(0)
    dinv = _dinv_from(deg_ref)
    agg = p_ref[0] + p_ref[1]
    h = dinv * (agg + u_ref[...]) + bias_ref[...]
    oh = _onehot(b_ref)

    @pl.when(i == 0)
    def _():
        pool_scr[...] = jnp.zeros_like(pool_scr)

    pool_scr[...] += jnp.dot(oh, h, preferred_element_type=jnp.float32)

    @pl.when(i == NBLK - 1)
    def _():
        cnt = jnp.maximum(cnt_ref[...][:, 0:1], 1.0)
        poolcat = jnp.concatenate(
            [pool1_ref[...], pool2_ref[...], pool_scr[...]], axis=1) / cnt
        t = jnp.dot(poolcat, w1_ref[...],
                    preferred_element_type=jnp.float32) + b1_ref[...]
        t = jnp.maximum(t, 0.0)
        o = jnp.dot(t, w2_ref[...],
                    preferred_element_type=jnp.float32) + b2_ref[...]
        h_ref[...] = o
        colid = lax.broadcasted_iota(jnp.int32, (G, 128), 1)
        valid = colid < OUT
        om = jnp.where(valid, o, -1e30)
        m = jnp.max(om, axis=1, keepdims=True)
        ssum = jnp.sum(jnp.where(valid, jnp.exp(om - m), 0.0),
                       axis=1, keepdims=True)
        ls_ref[...] = om - m - jnp.log(ssum)


def _last_call(p, u, deg2, bias, batch3, pool1, pool2, counts,
               lin1_W, lin1_b, lin2_Wp, lin2_bp):
    HH = 3 * H
    return pl.pallas_call(
        _last_body,
        grid=(NBLK,),
        in_specs=[
            pl.BlockSpec((NC, BLK, H), lambda i: (0, i, 0)),
            pl.BlockSpec((BLK, H), lambda i: (i, 0)),
            pl.BlockSpec((BLK, 2), lambda i: (i, 0)),
            pl.BlockSpec((1, H), lambda i: (0, 0)),
            pl.BlockSpec((1, 1, BLK), lambda i: (i, 0, 0)),
            pl.BlockSpec((G, H), lambda i: (0, 0)),
            pl.BlockSpec((G, H), lambda i: (0, 0)),
            pl.BlockSpec((G, G), lambda i: (0, 0)),
            pl.BlockSpec((HH, HH), lambda i: (0, 0)),
            pl.BlockSpec((1, HH), lambda i: (0, 0)),
            pl.BlockSpec((HH, 128), lambda i: (0, 0)),
            pl.BlockSpec((1, 128), lambda i: (0, 0)),
        ],
        out_specs=[
            pl.BlockSpec((G, 128), lambda i: (0, 0)),
            pl.BlockSpec((G, 128), lambda i: (0, 0)),
        ],
        out_shape=[
            jax.ShapeDtypeStruct((G, 128), jnp.float32),
            jax.ShapeDtypeStruct((G, 128), jnp.float32),
        ],
        scratch_shapes=[pltpu.VMEM((G, H), jnp.float32)],
    )(p, u, deg2, bias, batch3, pool1, pool2, counts,
      lin1_W, lin1_b, lin2_Wp, lin2_bp)


def kernel(x, edge_index, batch, W1, b1, W2, b2, W3, b3,
           lin1_W, lin1_b, lin2_W, lin2_b):
    f32 = jnp.float32
    row2d = edge_index[0].reshape(NW, NCH, CH)
    col2d = edge_index[1].reshape(NW, NCH, CH)
    col4d = edge_index[1].reshape(NW, NCH, 1, CH)
    x_pad = jnp.pad(x, ((0, N_PAD - N), (0, 0)))
    batch_pad = jnp.concatenate([batch, jnp.full((N_PAD - N,), G, jnp.int32)])
    batch3 = batch_pad.reshape(NBLK, 1, BLK)
    zeros1 = jnp.zeros((N_PAD,), f32)
    zeros2 = jnp.zeros((RPT, H), f32)

    degT = _deg_call()(col2d, zeros1)
    deg2 = degT.T

    u1, counts = _first_call(x_pad, W1, deg2, batch3)
    p = _agg_call()(u1, row2d, col4d, zeros2)
    u2, pool1 = _layer_call(p, u1, deg2, b1.reshape(1, H), W2, batch3)
    p = _agg_call()(u2, row2d, col4d, zeros2)
    u3, pool2 = _layer_call(p, u2, deg2, b2.reshape(1, H), W3, batch3)
    p = _agg_call()(u3, row2d, col4d, zeros2)
    lin2_Wp = jnp.pad(lin2_W, ((0, 0), (0, 128 - OUT)))
    lin2_bp = jnp.pad(lin2_b, (0, 128 - OUT)).reshape(1, 128)
    hout, ls = _last_call(p, u3, deg2, b3.reshape(1, H), batch3,
                          pool1, pool2, counts,
                          lin1_W, lin1_b.reshape(1, 3 * H), lin2_Wp, lin2_bp)
    return (hout[:, :OUT], ls[:, :OUT])
